# baseline (speedup 1.0000x reference)
import functools

import jax
import jax.numpy as jnp
from jax import lax
from jax.experimental import pallas as pl
from jax.experimental.pallas import tpu as pltpu
from jax.experimental.pallas import tpu_sc as plsc

B, S, D = 2, 2048, 1024
E, K = 64, 2
DFF = 2048
T = B * S
TK = T * K
C = int(2.0 * TK / E)
EC = E * C

GH = E // 2
GC = GH * C

NC, NS = 2, 16
NW = NC * NS
TOK_W = T // NW
TCH = 32
NCH = TOK_W // TCH


def _router_body(flat_ref, rw_ref, rb_ref, comb_ref, g_ref, mask_ref,
                 cg0_ref, cg1_ref):
    flat = flat_ref[...]
    logits = jnp.dot(flat, rw_ref[...], preferred_element_type=jnp.float32)
    logits = logits + rb_ref[...]
    lane = lax.broadcasted_iota(jnp.int32, (T, E), 1)
    v0 = jnp.max(logits, axis=1, keepdims=True)
    i0 = jnp.min(jnp.where(logits == v0, lane, E), axis=1, keepdims=True)
    l2 = jnp.where(lane == i0, -jnp.inf, logits)
    v1 = jnp.max(l2, axis=1, keepdims=True)
    i1 = jnp.min(jnp.where(l2 == v1, lane, E), axis=1, keepdims=True)
    e1 = jnp.exp(v1 - v0)
    denom = 1.0 + e1
    g_ref[:, 0:1] = 1.0 / denom
    g_ref[:, 1:2] = e1 / denom
    oh0 = (lane == i0).astype(jnp.float32)
    oh1 = (lane == i1).astype(jnp.float32)
    ssum = oh0 + oh1
    BLK = 256
    ri = lax.broadcasted_iota(jnp.int32, (BLK, BLK), 0)
    ci = lax.broadcasted_iota(jnp.int32, (BLK, BLK), 1)
    tri = (ci < ri).astype(jnp.float32)
    carry = jnp.zeros((1, E), jnp.float32)
    p0, p1 = [], []
    for b in range(T // BLK):
        blk = ssum[b * BLK:(b + 1) * BLK, :]
        excl = jnp.dot(tri, blk, preferred_element_type=jnp.float32) + carry
        p0.append(jnp.sum(excl * oh0[b * BLK:(b + 1) * BLK, :], axis=1, keepdims=True))
        p1.append(jnp.sum(excl * oh1[b * BLK:(b + 1) * BLK, :], axis=1, keepdims=True))
        carry = carry + jnp.sum(blk, axis=0, keepdims=True)
    pos0 = jnp.concatenate(p0, axis=0).astype(jnp.int32)
    pos1 = jnp.concatenate(p1, axis=0).astype(jnp.int32)
    cmin = jnp.min(carry)
    lane1 = lax.broadcasted_iota(jnp.int32, (1, E), 1)
    emin = jnp.min(jnp.where(carry == cmin, lane1, E))
    zrow = emin * C + (C - 1)
    slot0 = i0 * C + pos0
    slot1 = i1 * C + pos1
    c0 = jnp.where(pos0 < C, slot0, zrow)
    c1 = jnp.where(pos1 < C, slot1, zrow)
    comb_ref[:, 0:1] = c0
    comb_ref[:, 1:2] = c1
    cg0_ref[:, 0:1] = jnp.where(c0 < GC, c0, GC)
    cg0_ref[:, 1:2] = jnp.where(c1 < GC, c1, GC)
    cg1_ref[:, 0:1] = jnp.where(c0 >= GC, c0 - GC, GC)
    cg1_ref[:, 1:2] = jnp.where(c1 >= GC, c1 - GC, GC)
    ones = jnp.ones((T, 1), jnp.float32)
    cnt_col = lax.dot_general(ssum, ones, (((0,), (0,)), ((), ())),
                              preferred_element_type=jnp.float32)
    slot_iota = lax.broadcasted_iota(jnp.int32, (E, C), 1).astype(jnp.float32)
    mask_ref[...] = (slot_iota < cnt_col).astype(jnp.float32)


def _router_call(flat, router_w, router_b):
    return pl.pallas_call(
        _router_body,
        out_shape=[
            jax.ShapeDtypeStruct((T, K), jnp.int32),
            jax.ShapeDtypeStruct((T, K), jnp.float32),
            jax.ShapeDtypeStruct((E, C), jnp.float32),
            jax.ShapeDtypeStruct((T, K), jnp.int32),
            jax.ShapeDtypeStruct((T, K), jnp.int32),
        ],
    )(flat, router_w, router_b)


def _dispatch_body(flat_hbm, comb_hbm, gsc_hbm, buf_hbm, gw_hbm,
                   cfull, gfull, gw_tab, rows, idx0, idx1, semg, sems):
    wid = lax.axis_index("s") * NC + lax.axis_index("c")
    tbase = wid * TOK_W
    pltpu.sync_copy(comb_hbm, cfull)
    pltpu.sync_copy(gsc_hbm, gfull)
    gets = [None] * NCH
    for c in range(2):
        gets[c] = pltpu.async_copy(
            flat_hbm.at[pl.ds(tbase + c * TCH, TCH)], rows[c % 2], semg[c % 2])

    def scat_body(i, _):
        o = i * 16
        idx = cfull[pl.ds(o, 16)]
        plsc.store_scatter(gw_tab, [idx], gfull[pl.ds(o, 16)])
        return 0

    lax.fori_loop(0, TK // 16, scat_body, 0)

    lane = lax.broadcasted_iota(jnp.int32, (16,), 0)
    puts = [None] * NCH
    for c in range(NCH):
        p = c % 2
        jb = (tbase + c * TCH) * K
        for h in range(TCH // 16):
            idx0[p][pl.ds(h * 16, 16)] = plsc.load_gather(
                cfull, [jb + 2 * (h * 16 + lane)])
            idx1[p][pl.ds(h * 16, 16)] = plsc.load_gather(
                cfull, [jb + 2 * (h * 16 + lane) + 1])
        gets[c].wait()
        puts[c] = (
            pltpu.async_copy(rows[p], buf_hbm.at[idx0[p]], sems[p]),
            pltpu.async_copy(rows[p], buf_hbm.at[idx1[p]], sems[p]),
        )
        if c + 2 < NCH:
            puts[c][0].wait()
            puts[c][1].wait()
            puts[c] = None
            gets[c + 2] = pltpu.async_copy(
                flat_hbm.at[pl.ds(tbase + (c + 2) * TCH, TCH)], rows[p], semg[p])
    for c in range(NCH):
        if puts[c] is not None:
            puts[c][0].wait()
            puts[c][1].wait()
    pltpu.sync_copy(gw_tab.at[pl.ds(wid * (EC // NW), EC // NW)],
                    gw_hbm.at[pl.ds(wid * (EC // NW), EC // NW)])


@functools.lru_cache(maxsize=None)
def _dispatch_kernel():
    return pl.kernel(
        _dispatch_body,
        out_type=[
            jax.ShapeDtypeStruct((EC, D), jnp.float32),
            jax.ShapeDtypeStruct((EC,), jnp.float32),
        ],
        mesh=plsc.VectorSubcoreMesh(core_axis_name="c", subcore_axis_name="s",
                                    num_cores=NC, num_subcores=NS),
        compiler_params=pltpu.CompilerParams(needs_layout_passes=False),
        scratch_types=[
            pltpu.VMEM((TK,), jnp.int32),
            pltpu.VMEM((TK,), jnp.float32),
            pltpu.VMEM((EC,), jnp.float32),
            [pltpu.VMEM((TCH, D), jnp.float32)] * 2,
            [pltpu.VMEM((TCH,), jnp.int32)] * 2,
            [pltpu.VMEM((TCH,), jnp.int32)] * 2,
            [pltpu.SemaphoreType.DMA] * 2,
            [pltpu.SemaphoreType.DMA] * 2,
        ],
    )


def _ffn_body(buf_ref, w1_ref, w2_ref, gw_ref, m_ref, yw_ref):
    xb = buf_ref[0]
    h = jax.nn.gelu(jnp.dot(xb, w1_ref[0], preferred_element_type=jnp.float32))
    y = jnp.dot(h, w2_ref[0], preferred_element_type=jnp.float32)
    live = jnp.logical_and(m_ref[0] > 0, pl.program_id(0) < GH)
    yw_ref[0] = jnp.where(live, y * gw_ref[0], 0.0)


def _ffn_call(buf3, w1, w2, gw3, m3, base):
    def imap(e):
        return (jnp.minimum(e, GH - 1) + base, 0, 0)

    return pl.pallas_call(
        _ffn_body,
        grid=(GH + 1,),
        in_specs=[
            pl.BlockSpec((1, C, D), imap),
            pl.BlockSpec((1, D, DFF), imap),
            pl.BlockSpec((1, DFF, D), imap),
            pl.BlockSpec((1, C, 1), imap),
            pl.BlockSpec((1, C, 1), imap),
        ],
        out_specs=pl.BlockSpec((1, C, D), lambda e: (e, 0, 0)),
        out_shape=jax.ShapeDtypeStruct((GH + 1, C, D), jnp.float32),
    )(buf3, w1, w2, gw3, m3)


CT = 16
NCC = TOK_W // CT


def _combine_body(yw_hbm, comb_hbm, out_hbm, cidx, rows, semg, semp):
    wid = lax.axis_index("s") * NC + lax.axis_index("c")
    tbase = wid * TOK_W
    pltpu.sync_copy(comb_hbm.at[pl.ds(tbase * K, TOK_W * K)], cidx)
    NB = 3
    H = CT * K // 2

    def fire_get(c):
        p = c % NB
        return (
            pltpu.async_copy(yw_hbm.at[cidx.at[pl.ds(c * CT * K, H)]],
                             rows[p].at[pl.ds(0, H)], semg[p]),
            pltpu.async_copy(yw_hbm.at[cidx.at[pl.ds(c * CT * K + H, H)]],
                             rows[p].at[pl.ds(H, H)], semg[p]),
        )

    gets = [None] * NCC
    puts = [None] * NCC
    for c in range(2):
        gets[c] = fire_get(c)
    for c in range(NCC):
        p = c % NB
        gets[c][0].wait()
        gets[c][1].wait()

        def row_body(r, _2, _p=p):
            for q in range(D // 16):
                sl = pl.ds(q * 16, 16)
                rows[_p][r, sl] = rows[_p][2 * r, sl] + rows[_p][2 * r + 1, sl]
            return 0

        lax.fori_loop(0, CT, row_body, 0)
        puts[c] = pltpu.async_copy(rows[p].at[pl.ds(0, CT)],
                                   out_hbm.at[pl.ds(tbase + c * CT, CT)], semp[p])
        if c + 2 < NCC:
            if c >= 1:
                puts[c - 1].wait()
                puts[c - 1] = None
            gets[c + 2] = fire_get(c + 2)
    for c in range(NCC):
        if puts[c] is not None:
            puts[c].wait()


@functools.lru_cache(maxsize=None)
def _combine_kernel():
    return pl.kernel(
        _combine_body,
        out_type=jax.ShapeDtypeStruct((T, D), jnp.float32),
        mesh=plsc.VectorSubcoreMesh(core_axis_name="c", subcore_axis_name="s",
                                    num_cores=NC, num_subcores=NS),
        compiler_params=pltpu.CompilerParams(needs_layout_passes=False),
        scratch_types=[
            pltpu.VMEM((TOK_W * K,), jnp.int32),
            [pltpu.VMEM((2 * CT, D), jnp.float32)] * 3,
            [pltpu.SemaphoreType.DMA] * 3,
            [pltpu.SemaphoreType.DMA] * 3,
        ],
    )


def _combine2_body(yw_hbm, comb_hbm, part_hbm, out_hbm,
                   cidx, rows, part, semg, sempt, semp):
    wid = lax.axis_index("s") * NC + lax.axis_index("c")
    tbase = wid * TOK_W
    pltpu.sync_copy(comb_hbm.at[pl.ds(tbase * K, TOK_W * K)], cidx)
    H = CT * K // 2

    def fire_get(c):
        p = c % 2
        return (
            pltpu.async_copy(yw_hbm.at[cidx.at[pl.ds(c * CT * K, H)]],
                             rows[p].at[pl.ds(0, H)], semg[p]),
            pltpu.async_copy(yw_hbm.at[cidx.at[pl.ds(c * CT * K + H, H)]],
                             rows[p].at[pl.ds(H, H)], semg[p]),
        )

    def fire_pget(c):
        pp = c % 3
        return pltpu.async_copy(part_hbm.at[pl.ds(tbase + c * CT, CT)],
                                part[pp], sempt[pp])

    gets = [None] * NCC
    pgets = [None] * NCC
    puts = [None] * NCC
    for c in range(2):
        gets[c] = fire_get(c)
        pgets[c] = fire_pget(c)
    for c in range(NCC):
        pr = c % 2
        pp = c % 3
        gets[c][0].wait()
        gets[c][1].wait()
        pgets[c].wait()

        def row_body(r, _2, _pr=pr, _pp=pp):
            for q in range(D // 16):
                sl = pl.ds(q * 16, 16)
                part[_pp][r, sl] = (part[_pp][r, sl]
                                    + rows[_pr][2 * r, sl]
                                    + rows[_pr][2 * r + 1, sl])
            return 0

        lax.fori_loop(0, CT, row_body, 0)
        if c + 2 < NCC:
            gets[c + 2] = fire_get(c + 2)
        puts[c] = pltpu.async_copy(part[pp], out_hbm.at[pl.ds(tbase + c * CT, CT)],
                                   semp[pp])
        if c + 2 < NCC:
            if c >= 1:
                puts[c - 1].wait()
                puts[c - 1] = None
            pgets[c + 2] = fire_pget(c + 2)
    for c in range(NCC):
        if puts[c] is not None:
            puts[c].wait()


@functools.lru_cache(maxsize=None)
def _combine2_kernel():
    return pl.kernel(
        _combine2_body,
        out_type=jax.ShapeDtypeStruct((T, D), jnp.float32),
        mesh=plsc.VectorSubcoreMesh(core_axis_name="c", subcore_axis_name="s",
                                    num_cores=NC, num_subcores=NS),
        compiler_params=pltpu.CompilerParams(needs_layout_passes=False),
        scratch_types=[
            pltpu.VMEM((TOK_W * K,), jnp.int32),
            [pltpu.VMEM((2 * CT, D), jnp.float32)] * 2,
            [pltpu.VMEM((CT, D), jnp.float32)] * 3,
            [pltpu.SemaphoreType.DMA] * 2,
            [pltpu.SemaphoreType.DMA] * 3,
            [pltpu.SemaphoreType.DMA] * 3,
        ],
    )


def kernel(hidden_states, router_w, router_b, w1, w2):
    flat = hidden_states.reshape(T, D)
    comb, g, rowmask, cg0, cg1 = _router_call(flat, router_w,
                                              router_b.reshape(1, E))
    buf, gw = _dispatch_kernel()(flat, comb.reshape(TK), g.reshape(TK))
    buf3 = buf.reshape(E, C, D)
    gw3 = gw.reshape(E, C, 1)
    m3 = rowmask.reshape(E, C, 1)
    yw0 = _ffn_call(buf3, w1, w2, gw3, m3, 0)
    part = _combine_kernel()(yw0.reshape((GH + 1) * C, D), cg0.reshape(TK))
    yw1 = _ffn_call(buf3, w1, w2, gw3, m3, GH)
    out = _combine2_kernel()(yw1.reshape((GH + 1) * C, D), cg1.reshape(TK), part)
    return out.reshape(B, S, D)


# speedup vs baseline: 1.7814x; 1.7814x over previous
import functools

import jax
import jax.numpy as jnp
from jax import lax
from jax.experimental import pallas as pl
from jax.experimental.pallas import tpu as pltpu
from jax.experimental.pallas import tpu_sc as plsc

B, S, D = 2, 2048, 1024
E, K = 64, 2
DFF = 2048
T = B * S
TK = T * K
C = int(2.0 * TK / E)
EC = E * C

NC, NS = 2, 16
NW = NC * NS
TOK_W = T // NW
TCH = 32
NCH = TOK_W // TCH


def _router_body(flat_ref, rw_ref, rb_ref, comb_ref, g_ref, mask_ref):
    flat = flat_ref[...]
    logits = jnp.dot(flat, rw_ref[...], preferred_element_type=jnp.float32)
    logits = logits + rb_ref[...]
    lane = lax.broadcasted_iota(jnp.int32, (T, E), 1)
    v0 = jnp.max(logits, axis=1, keepdims=True)
    i0 = jnp.min(jnp.where(logits == v0, lane, E), axis=1, keepdims=True)
    l2 = jnp.where(lane == i0, -jnp.inf, logits)
    v1 = jnp.max(l2, axis=1, keepdims=True)
    i1 = jnp.min(jnp.where(l2 == v1, lane, E), axis=1, keepdims=True)
    e1 = jnp.exp(v1 - v0)
    denom = 1.0 + e1
    g_ref[:, 0:1] = 1.0 / denom
    g_ref[:, 1:2] = e1 / denom
    oh0 = (lane == i0).astype(jnp.float32)
    oh1 = (lane == i1).astype(jnp.float32)
    ssum = oh0 + oh1
    BLK = 256
    ri = lax.broadcasted_iota(jnp.int32, (BLK, BLK), 0)
    ci = lax.broadcasted_iota(jnp.int32, (BLK, BLK), 1)
    tri = (ci < ri).astype(jnp.float32)
    carry = jnp.zeros((1, E), jnp.float32)
    p0, p1 = [], []
    for b in range(T // BLK):
        blk = ssum[b * BLK:(b + 1) * BLK, :]
        excl = jnp.dot(tri, blk, preferred_element_type=jnp.float32) + carry
        p0.append(jnp.sum(excl * oh0[b * BLK:(b + 1) * BLK, :], axis=1, keepdims=True))
        p1.append(jnp.sum(excl * oh1[b * BLK:(b + 1) * BLK, :], axis=1, keepdims=True))
        carry = carry + jnp.sum(blk, axis=0, keepdims=True)
    pos0 = jnp.concatenate(p0, axis=0).astype(jnp.int32)
    pos1 = jnp.concatenate(p1, axis=0).astype(jnp.int32)
    cmin = jnp.min(carry)
    lane1 = lax.broadcasted_iota(jnp.int32, (1, E), 1)
    emin = jnp.min(jnp.where(carry == cmin, lane1, E))
    zrow = emin * C + (C - 1)
    slot0 = i0 * C + pos0
    slot1 = i1 * C + pos1
    comb_ref[:, 0:1] = jnp.where(pos0 < C, slot0, zrow)
    comb_ref[:, 1:2] = jnp.where(pos1 < C, slot1, zrow)
    ones = jnp.ones((T, 1), jnp.float32)
    cnt_col = lax.dot_general(ssum, ones, (((0,), (0,)), ((), ())),
                              preferred_element_type=jnp.float32)
    slot_iota = lax.broadcasted_iota(jnp.int32, (E, C), 1).astype(jnp.float32)
    mask_ref[...] = (slot_iota < cnt_col).astype(jnp.float32)


def _router_call(flat, router_w, router_b):
    return pl.pallas_call(
        _router_body,
        out_shape=[
            jax.ShapeDtypeStruct((T, K), jnp.int32),
            jax.ShapeDtypeStruct((T, K), jnp.float32),
            jax.ShapeDtypeStruct((E, C), jnp.float32),
        ],
    )(flat, router_w, router_b)


def _dispatch_body(flat_hbm, comb_hbm, gsc_hbm, buf_hbm, gw_hbm,
                   cfull, gfull, gw_tab, rows, idx0, idx1, semg, sems):
    wid = lax.axis_index("s") * NC + lax.axis_index("c")
    tbase = wid * TOK_W
    pltpu.sync_copy(comb_hbm, cfull)
    pltpu.sync_copy(gsc_hbm, gfull)
    gets = [None] * NCH
    for c in range(2):
        gets[c] = pltpu.async_copy(
            flat_hbm.at[pl.ds(tbase + c * TCH, TCH)], rows[c % 2], semg[c % 2])

    def scat_body(i, _):
        o = i * 16
        idx = cfull[pl.ds(o, 16)]
        plsc.store_scatter(gw_tab, [idx], gfull[pl.ds(o, 16)])
        return 0

    lax.fori_loop(0, TK // 16, scat_body, 0)

    lane = lax.broadcasted_iota(jnp.int32, (16,), 0)
    puts = [None] * NCH
    for c in range(NCH):
        p = c % 2
        jb = (tbase + c * TCH) * K
        for h in range(TCH // 16):
            idx0[p][pl.ds(h * 16, 16)] = plsc.load_gather(
                cfull, [jb + 2 * (h * 16 + lane)])
            idx1[p][pl.ds(h * 16, 16)] = plsc.load_gather(
                cfull, [jb + 2 * (h * 16 + lane) + 1])
        gets[c].wait()
        puts[c] = (
            pltpu.async_copy(rows[p], buf_hbm.at[idx0[p]], sems[p]),
            pltpu.async_copy(rows[p], buf_hbm.at[idx1[p]], sems[p]),
        )
        if c + 2 < NCH:
            puts[c][0].wait()
            puts[c][1].wait()
            puts[c] = None
            gets[c + 2] = pltpu.async_copy(
                flat_hbm.at[pl.ds(tbase + (c + 2) * TCH, TCH)], rows[p], semg[p])
    for c in range(NCH):
        if puts[c] is not None:
            puts[c][0].wait()
            puts[c][1].wait()
    pltpu.sync_copy(gw_tab.at[pl.ds(wid * (EC // NW), EC // NW)],
                    gw_hbm.at[pl.ds(wid * (EC // NW), EC // NW)])


@functools.lru_cache(maxsize=None)
def _dispatch_kernel():
    return pl.kernel(
        _dispatch_body,
        out_type=[
            jax.ShapeDtypeStruct((EC, D), jnp.float32),
            jax.ShapeDtypeStruct((EC,), jnp.float32),
        ],
        mesh=plsc.VectorSubcoreMesh(core_axis_name="c", subcore_axis_name="s",
                                    num_cores=NC, num_subcores=NS),
        compiler_params=pltpu.CompilerParams(needs_layout_passes=False),
        scratch_types=[
            pltpu.VMEM((TK,), jnp.int32),
            pltpu.VMEM((TK,), jnp.float32),
            pltpu.VMEM((EC,), jnp.float32),
            [pltpu.VMEM((TCH, D), jnp.float32)] * 2,
            [pltpu.VMEM((TCH,), jnp.int32)] * 2,
            [pltpu.VMEM((TCH,), jnp.int32)] * 2,
            [pltpu.SemaphoreType.DMA] * 2,
            [pltpu.SemaphoreType.DMA] * 2,
        ],
    )


def _ffn_body(buf_ref, w1_ref, w2_ref, gw_ref, m_ref, yw_ref):
    xb = buf_ref[0]
    h = jax.nn.gelu(jnp.dot(xb, w1_ref[0], preferred_element_type=jnp.float32))
    y = jnp.dot(h, w2_ref[0], preferred_element_type=jnp.float32)
    yw_ref[0] = jnp.where(m_ref[0] > 0, y * gw_ref[0], 0.0)


def _ffn_call(buf3, w1, w2, gw3, m3):
    return pl.pallas_call(
        _ffn_body,
        grid=(E,),
        in_specs=[
            pl.BlockSpec((1, C, D), lambda e: (e, 0, 0)),
            pl.BlockSpec((1, D, DFF), lambda e: (e, 0, 0)),
            pl.BlockSpec((1, DFF, D), lambda e: (e, 0, 0)),
            pl.BlockSpec((1, C, 1), lambda e: (e, 0, 0)),
            pl.BlockSpec((1, C, 1), lambda e: (e, 0, 0)),
        ],
        out_specs=pl.BlockSpec((1, C, D), lambda e: (e, 0, 0)),
        out_shape=jax.ShapeDtypeStruct((E, C, D), jnp.float32),
    )(buf3, w1, w2, gw3, m3)


CT = 16
NCC = TOK_W // CT


def _combine_body(yw_hbm, comb_hbm, out_hbm, cidx, rows, semg, semp):
    wid = lax.axis_index("s") * NC + lax.axis_index("c")
    tbase = wid * TOK_W
    pltpu.sync_copy(comb_hbm.at[pl.ds(tbase * K, TOK_W * K)], cidx)
    NB = 3
    H = CT * K // 2

    def fire_get(c):
        p = c % NB
        return (
            pltpu.async_copy(yw_hbm.at[cidx.at[pl.ds(c * CT * K, H)]],
                             rows[p].at[pl.ds(0, H)], semg[p]),
            pltpu.async_copy(yw_hbm.at[cidx.at[pl.ds(c * CT * K + H, H)]],
                             rows[p].at[pl.ds(H, H)], semg[p]),
        )

    gets = [None] * NCC
    puts = [None] * NCC
    for c in range(2):
        gets[c] = fire_get(c)
    for c in range(NCC):
        p = c % NB
        gets[c][0].wait()
        gets[c][1].wait()

        def row_body(r, _2, _p=p):
            for q in range(D // 16):
                sl = pl.ds(q * 16, 16)
                rows[_p][r, sl] = rows[_p][2 * r, sl] + rows[_p][2 * r + 1, sl]
            return 0

        lax.fori_loop(0, CT, row_body, 0)
        puts[c] = pltpu.async_copy(rows[p].at[pl.ds(0, CT)],
                                   out_hbm.at[pl.ds(tbase + c * CT, CT)], semp[p])
        if c + 2 < NCC:
            if c >= 1:
                puts[c - 1].wait()
                puts[c - 1] = None
            gets[c + 2] = fire_get(c + 2)
    for c in range(NCC):
        if puts[c] is not None:
            puts[c].wait()


@functools.lru_cache(maxsize=None)
def _combine_kernel():
    return pl.kernel(
        _combine_body,
        out_type=jax.ShapeDtypeStruct((T, D), jnp.float32),
        mesh=plsc.VectorSubcoreMesh(core_axis_name="c", subcore_axis_name="s",
                                    num_cores=NC, num_subcores=NS),
        compiler_params=pltpu.CompilerParams(needs_layout_passes=False),
        scratch_types=[
            pltpu.VMEM((TOK_W * K,), jnp.int32),
            [pltpu.VMEM((2 * CT, D), jnp.float32)] * 3,
            [pltpu.SemaphoreType.DMA] * 3,
            [pltpu.SemaphoreType.DMA] * 3,
        ],
    )


def kernel(hidden_states, router_w, router_b, w1, w2):
    flat = hidden_states.reshape(T, D)
    comb, g, rowmask = _router_call(flat, router_w, router_b.reshape(1, E))
    buf, gw = _dispatch_kernel()(flat, comb.reshape(TK), g.reshape(TK))
    yw = _ffn_call(buf.reshape(E, C, D), w1, w2, gw.reshape(E, C, 1),
                   rowmask.reshape(E, C, 1))
    out = _combine_kernel()(yw.reshape(EC, D), comb.reshape(TK))
    return out.reshape(B, S, D)
